# baseline (speedup 1.0000x reference)
import functools

import jax
import jax.numpy as jnp
from jax.experimental import pallas as pl
from jax.experimental.pallas import tpu as pltpu

CPAD = 128


def _fused_kernel(x_ref, bm_ref, wcat_ref, bcat_ref, w2cat_ref, b2_ref,
                  bp2_ref, o_ref, *, H, E, PH):
    x = x_ref[...]
    h = jnp.dot(x, wcat_ref[...], preferred_element_type=jnp.float32)
    h = jnp.maximum(h + bcat_ref[...], 0.0)
    bm = bm_ref[...]
    bn = h.shape[0]
    hp = h[:, H:].reshape(bn, E, PH) * bm[:, :, None]
    hm = jnp.concatenate([h[:, :H], hp.reshape(bn, E * PH)], axis=1)
    o = jnp.dot(hm, w2cat_ref[...], preferred_element_type=jnp.float32)
    o = o + b2_ref[...] + jnp.dot(bm, bp2_ref[...],
                                  preferred_element_type=jnp.float32)
    o_ref[...] = o


def kernel(x, in_bitmap, W1, b1, W2, b2, Wp1, bp1, Wp2, bp2):
    N, D = x.shape
    H = W1.shape[1]
    E, _, PH = Wp1.shape
    C = W2.shape[1]
    F = H + E * PH

    Wcat = jnp.concatenate([W1, Wp1.transpose(1, 0, 2).reshape(D, E * PH)],
                           axis=1)
    bcat = jnp.concatenate([b1, bp1.reshape(-1)]).reshape(1, F)
    W2cat = jnp.zeros((F, CPAD), jnp.float32).at[:, :C].set(
        jnp.concatenate([W2, Wp2.reshape(E * PH, C)], axis=0))
    b2p = jnp.zeros((1, CPAD), jnp.float32).at[0, :C].set(b2)
    bp2p = jnp.zeros((E, CPAD), jnp.float32).at[:, :C].set(bp2)
    bm = in_bitmap.astype(jnp.float32)

    BN = 512
    grid = (N // BN,)
    out = pl.pallas_call(
        functools.partial(_fused_kernel, H=H, E=E, PH=PH),
        grid=grid,
        in_specs=[
            pl.BlockSpec((BN, D), lambda i: (i, 0)),
            pl.BlockSpec((BN, E), lambda i: (i, 0)),
            pl.BlockSpec((D, F), lambda i: (0, 0)),
            pl.BlockSpec((1, F), lambda i: (0, 0)),
            pl.BlockSpec((F, CPAD), lambda i: (0, 0)),
            pl.BlockSpec((1, CPAD), lambda i: (0, 0)),
            pl.BlockSpec((E, CPAD), lambda i: (0, 0)),
        ],
        out_specs=pl.BlockSpec((BN, CPAD), lambda i: (i, 0)),
        out_shape=jax.ShapeDtypeStruct((N, CPAD), jnp.float32),
        compiler_params=pltpu.CompilerParams(
            dimension_semantics=("arbitrary",),
        ),
    )(x, bm, Wcat, bcat, W2cat, b2p, bp2p)
    return out[:, :C]


# speedup vs baseline: 1.0686x; 1.0686x over previous
import functools

import jax
import jax.numpy as jnp
from jax.experimental import pallas as pl
from jax.experimental.pallas import tpu as pltpu

CPAD = 128


def _fused_kernel(x_ref, bm_ref, w1_ref, b1_ref, w2_ref, b2_ref,
                  wp1_ref, bp1_ref, wp2_ref, bp2_ref, o_ref, *, E):
    x = x_ref[...]
    bm = bm_ref[...]
    h = jnp.dot(x, w1_ref[...], preferred_element_type=jnp.float32)
    h = jnp.maximum(h + b1_ref[...], 0.0)
    o = jnp.dot(h, w2_ref[...], preferred_element_type=jnp.float32)
    for e in range(E):
        he = jnp.dot(x, wp1_ref[e], preferred_element_type=jnp.float32)
        he = jnp.maximum(he + bp1_ref[e], 0.0) * bm[:, e][:, None]
        o = o + jnp.dot(he, wp2_ref[e], preferred_element_type=jnp.float32)
    o = o + b2_ref[...] + jnp.dot(bm, bp2_ref[...],
                                  preferred_element_type=jnp.float32)
    o_ref[...] = o


def kernel(x, in_bitmap, W1, b1, W2, b2, Wp1, bp1, Wp2, bp2):
    N, D = x.shape
    H = W1.shape[1]
    E, _, PH = Wp1.shape
    C = W2.shape[1]

    xb = x.astype(jnp.bfloat16)
    W1b = W1.astype(jnp.bfloat16)
    Wp1b = Wp1.astype(jnp.bfloat16)
    W2p = jnp.zeros((H, CPAD), jnp.float32).at[:, :C].set(W2)
    Wp2p = jnp.zeros((E, PH, CPAD), jnp.float32).at[:, :, :C].set(Wp2)
    b2p = jnp.zeros((1, CPAD), jnp.float32).at[0, :C].set(b2)
    bp2p = jnp.zeros((E, CPAD), jnp.float32).at[:, :C].set(bp2)
    bm = in_bitmap.astype(jnp.float32)

    BN = 512
    grid = (N // BN,)
    out = pl.pallas_call(
        functools.partial(_fused_kernel, E=E),
        grid=grid,
        in_specs=[
            pl.BlockSpec((BN, D), lambda i: (i, 0)),
            pl.BlockSpec((BN, E), lambda i: (i, 0)),
            pl.BlockSpec((D, H), lambda i: (0, 0)),
            pl.BlockSpec((1, H), lambda i: (0, 0)),
            pl.BlockSpec((H, CPAD), lambda i: (0, 0)),
            pl.BlockSpec((1, CPAD), lambda i: (0, 0)),
            pl.BlockSpec((E, D, PH), lambda i: (0, 0, 0)),
            pl.BlockSpec((E, PH), lambda i: (0, 0)),
            pl.BlockSpec((E, PH, CPAD), lambda i: (0, 0, 0)),
            pl.BlockSpec((E, CPAD), lambda i: (0, 0)),
        ],
        out_specs=pl.BlockSpec((BN, CPAD), lambda i: (i, 0)),
        out_shape=jax.ShapeDtypeStruct((N, CPAD), jnp.float32),
        compiler_params=pltpu.CompilerParams(
            dimension_semantics=("arbitrary",),
        ),
    )(xb, bm, W1b, b1.reshape(1, H), W2p, b2p, Wp1b, bp1, Wp2p, bp2p)
    return out[:, :C]


# speedup vs baseline: 1.3099x; 1.2258x over previous
import functools

import jax
import jax.numpy as jnp
from jax.experimental import pallas as pl
from jax.experimental.pallas import tpu as pltpu

CPAD = 128


def _fused_kernel(x_ref, bm_ref, w1_ref, b1_ref, w2_ref, b2_ref,
                  wp1_ref, bp1_ref, wp2_ref, bp2_ref, o_ref, *, E):
    x = x_ref[...]
    bm = bm_ref[...]
    h = jnp.dot(x, w1_ref[...], preferred_element_type=jnp.float32)
    h = jnp.maximum(h + b1_ref[...], 0.0)
    o = jnp.dot(h, w2_ref[...], preferred_element_type=jnp.float32)
    for e in range(E):
        he = jnp.dot(x, wp1_ref[e], preferred_element_type=jnp.float32)
        he = jnp.maximum(he + bp1_ref[e], 0.0) * bm[:, e][:, None]
        o = o + jnp.dot(he, wp2_ref[e], preferred_element_type=jnp.float32)
    o = o + b2_ref[...] + jnp.dot(bm, bp2_ref[...],
                                  preferred_element_type=jnp.float32)
    o_ref[...] = o


def kernel(x, in_bitmap, W1, b1, W2, b2, Wp1, bp1, Wp2, bp2):
    N, D = x.shape
    H = W1.shape[1]
    E, _, PH = Wp1.shape
    C = W2.shape[1]

    bm = in_bitmap.astype(jnp.float32)

    BN = 512
    grid = (N // BN,)
    out = pl.pallas_call(
        functools.partial(_fused_kernel, E=E),
        grid=grid,
        in_specs=[
            pl.BlockSpec((BN, D), lambda i: (i, 0)),
            pl.BlockSpec((BN, E), lambda i: (i, 0)),
            pl.BlockSpec((D, H), lambda i: (0, 0)),
            pl.BlockSpec((1, H), lambda i: (0, 0)),
            pl.BlockSpec((H, C), lambda i: (0, 0)),
            pl.BlockSpec((1, C), lambda i: (0, 0)),
            pl.BlockSpec((E, D, PH), lambda i: (0, 0, 0)),
            pl.BlockSpec((E, PH), lambda i: (0, 0)),
            pl.BlockSpec((E, PH, C), lambda i: (0, 0, 0)),
            pl.BlockSpec((E, C), lambda i: (0, 0)),
        ],
        out_specs=pl.BlockSpec((BN, C), lambda i: (i, 0)),
        out_shape=jax.ShapeDtypeStruct((N, C), jnp.float32),
        compiler_params=pltpu.CompilerParams(
            dimension_semantics=("arbitrary",),
        ),
    )(x, bm, W1, b1.reshape(1, H), W2, b2.reshape(1, C), Wp1, bp1, Wp2, bp2)
    return out
